# baseline (device time: 311734 ns/iter reference)
import jax
import jax.numpy as jnp
from jax import lax
from jax.experimental import pallas as pl
from jax.experimental.pallas import tpu as pltpu

N_DEV = 16
N_QT = 4
_GELU_C = 0.7978845608028654

_SEND_UP = {
    0: [(0, 0, 1), (1, 0, 1), (2, 0, 1)],
    1: [(1, 1, 3), (2, 1, 2)],
    2: [(2, 3, 3)],
}
_SEND_DN = {
    0: [(1, 0, 1), (2, 0, 2), (3, 0, 2)],
    1: [(1, 2, 2), (2, 2, 3)],
    2: [(1, 3, 3)],
}
_RECV_UP = {0: [(1, 1), (2, 1), (3, 1)], 1: [(2, 3), (3, 2)], 2: [(3, 3)]}
_RECV_DN = {0: [(0, 1), (1, 2), (2, 2)], 1: [(0, 2), (1, 3)], 2: [(0, 3)]}

_QT_ORDER = (0, 2, 1, 3)


def _gelu(y):
    return 0.5 * y * (1.0 + jnp.tanh(_GELU_C * (y + 0.044715 * y * y * y)))


def kernel(x, w_mat):
    m_per, k_dim = x.shape
    _, n_per = w_mat.shape
    q = m_per // N_QT

    def body(x_ref, w_ref, out_ref, *s):
        stg = s[0:4]
        pbuf = s[4:8]
        up_send, up_recv, dn_send, dn_recv = s[8:12]
        h0send = s[12:16]
        psend = s[16:20]
        precv = s[20:24]
        cred = s[24:28]
        exit_sem = s[28]

        my = lax.axis_index("i")
        z = my // 4
        j = my % 4
        jr_id = 4 * z + (j + 1) % 4
        jl_id = 4 * z + (j - 1) % 4
        up_id = my + 4
        dn_id = my - 4

        def gemm_store(src_val, origin_id, row_off, nrows):
            y = jnp.dot(src_val, w_ref[:, :],
                        preferred_element_type=jnp.float32)
            out_ref[pl.ds(origin_id * m_per + row_off, nrows), :] = _gelu(y)

        def zdesc(qt, up, kk, src_slot, dst_slot, dev):
            ssem = up_send if up else dn_send
            rsem = up_recv if up else dn_recv
            return pltpu.make_async_remote_copy(
                src_ref=stg[qt].at[src_slot],
                dst_ref=stg[qt].at[dst_slot],
                send_sem=ssem.at[4 * kk + qt],
                recv_sem=rsem.at[4 * kk + qt],
                device_id=(dev,),
                device_id_type=pl.DeviceIdType.MESH,
            )

        def pdesc(t, qt):
            h, r = t % 3, t // 3
            cw = qt < 2
            dev = jr_id if cw else jl_id
            if h == 0:
                src, ssem = stg[qt].at[r], h0send[qt].at[r]
            else:
                src, ssem = pbuf[qt].at[(t - 1) % 3], psend[qt].at[t % 3]
            return pltpu.make_async_remote_copy(
                src_ref=src, dst_ref=pbuf[qt].at[t % 3],
                send_sem=ssem, recv_sem=precv[qt].at[t % 3],
                device_id=(dev,), device_id_type=pl.DeviceIdType.MESH,
            )

        def ozr(r):
            if r == 0:
                return z
            if r == 1:
                return jnp.where(z == 0, 1, z - 1)
            if r == 2:
                return jnp.where(z == 0, 2,
                                 jnp.where(z == 1, 2, jnp.where(z == 2, 3, 1)))
            return jnp.where(z <= 1, 3, 0)

        def z_sends(kk):
            for up, table, dev in ((True, _SEND_UP, up_id), (False, _SEND_DN, dn_id)):
                for zv, src_slot, dst_slot in table[kk]:
                    @pl.when(z == zv)
                    def _():
                        for qt in range(N_QT):
                            zdesc(qt, up, kk, src_slot, dst_slot, dev).start()

        def z_block(kk):
            for up, rtable, dev in ((True, _RECV_UP, dn_id), (False, _RECV_DN, up_id)):
                for zv, slot in rtable[kk]:
                    @pl.when(z == zv)
                    def _():
                        for qt in range(N_QT):
                            zdesc(qt, up, kk, 0, slot, dev).wait_recv()
            if kk + 1 in _SEND_UP:
                z_sends(kk + 1)
            for up, rtable in ((True, _RECV_UP), (False, _RECV_DN)):
                for zv, slot in rtable[kk]:
                    oid = 4 * ((zv - 1 - kk) if up else (zv + 1 + kk)) + j
                    @pl.when(z == zv)
                    def _():
                        for qt in range(N_QT):
                            gemm_store(stg[qt][slot, :, :], oid, qt * q, q)

        barrier_sem = pltpu.get_barrier_semaphore()
        for nbr in (jl_id, jr_id):
            pl.semaphore_signal(barrier_sem, inc=1, device_id=(nbr,),
                                device_id_type=pl.DeviceIdType.MESH)

        @pl.when(z < 3)
        def _():
            pl.semaphore_signal(barrier_sem, inc=1, device_id=(up_id,),
                                device_id_type=pl.DeviceIdType.MESH)

        @pl.when(z > 0)
        def _():
            pl.semaphore_signal(barrier_sem, inc=1, device_id=(dn_id,),
                                device_id_type=pl.DeviceIdType.MESH)

        pl.semaphore_wait(barrier_sem, 2)

        @pl.when(z < 3)
        def _():
            pl.semaphore_wait(barrier_sem, 1)

        @pl.when(z > 0)
        def _():
            pl.semaphore_wait(barrier_sem, 1)

        for qt in range(N_QT):
            stg[qt][0, :, :] = x_ref[pl.ds(qt * q, q), :]
        z_sends(0)

        for t in range(12):
            descs = {}
            for qt in _QT_ORDER:
                if t >= 1:
                    pdesc(t - 1, qt).wait_recv()
                if t >= 3:
                    pl.semaphore_wait(cred[qt], 1)
                descs[qt] = pdesc(t, qt)
                descs[qt].start()
            if t == 0:
                y = jnp.dot(x_ref[:, :], w_ref[:, :],
                            preferred_element_type=jnp.float32)
                out_ref[pl.ds(my * m_per, m_per), :] = _gelu(y)
            else:
                hp, rp = (t - 1) % 3, (t - 1) // 3
                ozp = ozr(rp)
                id_cw = 4 * ozp + (j - hp - 1) % 4
                id_ccw = 4 * ozp + (j + hp + 1) % 4
                for qt in range(N_QT):
                    oid = id_cw if qt < 2 else id_ccw
                    gemm_store(pbuf[qt][(t - 1) % 3, :, :], oid, qt * q, q)
            if t % 3 != 0:
                for qt in _QT_ORDER:
                    descs[qt].wait_send()
            if 1 <= t <= 9:
                for qt in range(N_QT):
                    pl.semaphore_signal(
                        cred[qt], inc=1,
                        device_id=(jl_id if qt < 2 else jr_id,),
                        device_id_type=pl.DeviceIdType.MESH)
            if t in (2, 5, 8):
                z_block(t // 3)

        oz3 = ozr(3)
        for qt in _QT_ORDER:
            pdesc(11, qt).wait_recv()
            oid = 4 * oz3 + ((j + 1) % 4 if qt < 2 else (j - 1) % 4)
            gemm_store(pbuf[qt][11 % 3, :, :], oid, qt * q, q)

        for r in range(4):
            for qt in range(N_QT):
                pdesc(3 * r, qt).wait_send()
        for kk in range(3):
            for up, table, dev in ((True, _SEND_UP, up_id), (False, _SEND_DN, dn_id)):
                for zv, src_slot, dst_slot in table[kk]:
                    @pl.when(z == zv)
                    def _():
                        for qt in range(N_QT):
                            zdesc(qt, up, kk, src_slot, dst_slot, dev).wait_send()

        for nbr in (jl_id, jr_id):
            pl.semaphore_signal(exit_sem, inc=1, device_id=(nbr,),
                                device_id_type=pl.DeviceIdType.MESH)

        @pl.when(z < 3)
        def _():
            pl.semaphore_signal(exit_sem, inc=1, device_id=(up_id,),
                                device_id_type=pl.DeviceIdType.MESH)

        @pl.when(z > 0)
        def _():
            pl.semaphore_signal(exit_sem, inc=1, device_id=(dn_id,),
                                device_id_type=pl.DeviceIdType.MESH)

        pl.semaphore_wait(exit_sem, 2)

        @pl.when(z < 3)
        def _():
            pl.semaphore_wait(exit_sem, 1)

        @pl.when(z > 0)
        def _():
            pl.semaphore_wait(exit_sem, 1)

    return pl.pallas_call(
        body,
        out_shape=jax.ShapeDtypeStruct((N_DEV * m_per, n_per), jnp.float32),
        in_specs=[
            pl.BlockSpec(memory_space=pltpu.VMEM),
            pl.BlockSpec(memory_space=pltpu.VMEM),
        ],
        out_specs=pl.BlockSpec(memory_space=pltpu.VMEM),
        scratch_shapes=(
            [pltpu.VMEM((4, q, k_dim), jnp.float32)] * 4
            + [pltpu.VMEM((3, q, k_dim), jnp.float32)] * 4
            + [pltpu.SemaphoreType.DMA((12,))] * 4
            + [pltpu.SemaphoreType.DMA((4,))] * 4
            + [pltpu.SemaphoreType.DMA((3,))] * 4
            + [pltpu.SemaphoreType.DMA((3,))] * 4
            + [pltpu.SemaphoreType.REGULAR] * 4
            + [pltpu.SemaphoreType.REGULAR]
        ),
        compiler_params=pltpu.CompilerParams(collective_id=0),
    )(x, w_mat)
